# baseline (device time: 8154 ns/iter reference)
import jax
import jax.numpy as jnp
from jax import lax
from jax.experimental import pallas as pl
from jax.experimental.pallas import tpu as pltpu

N_CHUNKS = 4
ROWS = 256 // N_CHUNKS


def kernel(x, pi):
    def body(x_ref, pi_ref, out_ref, xv_ref, comm_ref, recv_ref, pi_smem,
             x_sems, pi_sem, out_sems, fix_sem, send_sems, recv_sems):
        my_x = lax.axis_index("x")
        my_y = lax.axis_index("y")
        peer = (my_x, 1 - my_y)

        cp_pi = pltpu.make_async_copy(pi_ref, pi_smem, pi_sem)
        cp_pi.start()
        cp_x = []
        for c in range(N_CHUNKS):
            rows = pl.ds(c * ROWS, ROWS)
            cp = pltpu.make_async_copy(
                x_ref.at[0, rows, :], xv_ref.at[0, rows, :], x_sems.at[c]
            )
            cp.start()
            cp_x.append(cp)

        barrier = pltpu.get_barrier_semaphore()
        pl.semaphore_signal(
            barrier,
            inc=1,
            device_id=peer,
            device_id_type=pl.DeviceIdType.MESH,
        )
        pl.semaphore_wait(barrier, 1)

        rdmas = []
        for c in range(N_CHUNKS):
            rows = pl.ds(c * ROWS, ROWS)
            cp_x[c].wait()
            comm_ref[0, rows, :] = xv_ref[0, rows, :].astype(jnp.bfloat16)
            rdma = pltpu.make_async_remote_copy(
                src_ref=comm_ref.at[0, rows, :],
                dst_ref=recv_ref.at[0, rows, :],
                send_sem=send_sems.at[c],
                recv_sem=recv_sems.at[c],
                device_id=peer,
                device_id_type=pl.DeviceIdType.MESH,
            )
            rdma.start()
            rdmas.append(rdma)

        cp_out = []
        for c in range(N_CHUNKS):
            rows = pl.ds(c * ROWS, ROWS)
            rdmas[c].wait_recv()
            cp = pltpu.make_async_copy(
                recv_ref.at[0, rows, :], out_ref.at[0, rows, :], out_sems.at[c]
            )
            cp.start()
            cp_out.append(cp)
        for rdma in rdmas:
            rdma.wait_send()

        cp_pi.wait()
        is_identity = pi_smem[my_y] == my_y

        @pl.when(is_identity)
        def _():
            for cp in cp_out:
                cp.wait()
            cp_fix = pltpu.make_async_copy(comm_ref, out_ref, fix_sem)
            cp_fix.start()
            cp_fix.wait()

        @pl.when(jnp.logical_not(is_identity))
        def _():
            for cp in cp_out:
                cp.wait()

    x = pltpu.with_memory_space_constraint(x, pltpu.HBM)
    pi = pltpu.with_memory_space_constraint(pi, pltpu.HBM)
    return pl.pallas_call(
        body,
        out_shape=pltpu.HBM(x.shape, jnp.bfloat16),
        in_specs=[
            pl.BlockSpec(memory_space=pl.ANY),
            pl.BlockSpec(memory_space=pl.ANY),
        ],
        out_specs=pl.BlockSpec(memory_space=pl.ANY),
        scratch_shapes=[
            pltpu.VMEM((1, 256, 256), jnp.float32),
            pltpu.VMEM((1, 256, 256), jnp.bfloat16),
            pltpu.VMEM((1, 256, 256), jnp.bfloat16),
            pltpu.SMEM((2,), jnp.int32),
            pltpu.SemaphoreType.DMA((N_CHUNKS,)),
            pltpu.SemaphoreType.DMA,
            pltpu.SemaphoreType.DMA((N_CHUNKS,)),
            pltpu.SemaphoreType.DMA,
            pltpu.SemaphoreType.DMA((N_CHUNKS,)),
            pltpu.SemaphoreType.DMA((N_CHUNKS,)),
        ],
        compiler_params=pltpu.CompilerParams(collective_id=0),
    )(x, pi)


# device time: 6614 ns/iter; 1.2328x vs baseline; 1.2328x over previous
import jax
import jax.numpy as jnp
from jax import lax
from jax.experimental import pallas as pl
from jax.experimental.pallas import tpu as pltpu

N_CHUNKS = 4
ROWS = 256 // N_CHUNKS


def kernel(x, pi):
    def body(x_ref, pi_ref, out_ref, xv_ref, comm_ref, pi_smem,
             x_sems, pi_sem, fix_sem, send_sems, recv_sems):
        my_x = lax.axis_index("x")
        my_y = lax.axis_index("y")
        peer = (my_x, 1 - my_y)

        cp_pi = pltpu.make_async_copy(pi_ref, pi_smem, pi_sem)
        cp_pi.start()
        cp_x = []
        for c in range(N_CHUNKS):
            rows = pl.ds(c * ROWS, ROWS)
            cp = pltpu.make_async_copy(
                x_ref.at[0, rows, :], xv_ref.at[0, rows, :], x_sems.at[c]
            )
            cp.start()
            cp_x.append(cp)

        barrier = pltpu.get_barrier_semaphore()
        pl.semaphore_signal(
            barrier,
            inc=1,
            device_id=peer,
            device_id_type=pl.DeviceIdType.MESH,
        )

        rows0 = pl.ds(0, ROWS)
        cp_x[0].wait()
        comm_ref[0, rows0, :] = xv_ref[0, rows0, :].astype(jnp.bfloat16)

        pl.semaphore_wait(barrier, 1)

        rdmas = []
        for c in range(N_CHUNKS):
            rows = pl.ds(c * ROWS, ROWS)
            if c > 0:
                cp_x[c].wait()
                comm_ref[0, rows, :] = xv_ref[0, rows, :].astype(jnp.bfloat16)
            rdma = pltpu.make_async_remote_copy(
                src_ref=comm_ref.at[0, rows, :],
                dst_ref=out_ref.at[0, rows, :],
                send_sem=send_sems.at[c],
                recv_sem=recv_sems.at[c],
                device_id=peer,
                device_id_type=pl.DeviceIdType.MESH,
            )
            rdma.start()
            rdmas.append(rdma)
        for rdma in rdmas:
            rdma.wait()

        cp_pi.wait()

        @pl.when(pi_smem[my_y] == my_y)
        def _():
            cp_fix = pltpu.make_async_copy(comm_ref, out_ref, fix_sem)
            cp_fix.start()
            cp_fix.wait()

    x = pltpu.with_memory_space_constraint(x, pltpu.HBM)
    pi = pltpu.with_memory_space_constraint(pi, pltpu.HBM)
    return pl.pallas_call(
        body,
        out_shape=jax.ShapeDtypeStruct(x.shape, jnp.bfloat16),
        in_specs=[
            pl.BlockSpec(memory_space=pl.ANY),
            pl.BlockSpec(memory_space=pl.ANY),
        ],
        out_specs=pl.BlockSpec(memory_space=pl.ANY),
        scratch_shapes=[
            pltpu.VMEM((1, 256, 256), jnp.float32),
            pltpu.VMEM((1, 256, 256), jnp.bfloat16),
            pltpu.SMEM((2,), jnp.int32),
            pltpu.SemaphoreType.DMA((N_CHUNKS,)),
            pltpu.SemaphoreType.DMA,
            pltpu.SemaphoreType.DMA,
            pltpu.SemaphoreType.DMA((N_CHUNKS,)),
            pltpu.SemaphoreType.DMA((N_CHUNKS,)),
        ],
        compiler_params=pltpu.CompilerParams(collective_id=0),
    )(x, pi)
